# baseline (speedup 1.0000x reference)
import numpy as np

import jax
import jax.numpy as jnp
from jax import lax
from jax.experimental import pallas as pl
from jax.experimental.pallas import tpu as pltpu
from jax.experimental.pallas import tpu_sc as plsc

_ROWS = 128
_COLS = 100000
_W = 2048
_NCM = 33
_C0 = _NCM * _W
_SW = 32768
_NCS = _SW // _W
_K1 = 42

_X1_BASE = (
    np.arange(_ROWS, dtype=np.uint32)[:, None] * np.uint32(_COLS)
    + np.arange(_W, dtype=np.uint32)[None, :]
    + np.uint32(_K1)
)
_X1_END = np.broadcast_to(
    (np.arange(_ROWS, dtype=np.uint32)[:, None] + 1) * np.uint32(_COLS)
    + np.uint32(_K1),
    (_ROWS, _W),
).copy()


def _threefry_from_x1(x1):
    ks2 = 0x1BD11BDA ^ _K1
    rots = ((13, 15, 26, 6), (17, 29, 16, 24))
    c0 = (_K1, ks2, 0, _K1, ks2)
    c1 = (ks2 + 1, 2, _K1 + 3, ks2 + 4, 5)

    x0 = x1
    x1 = (x1 << jnp.uint32(13)) | (x1 >> jnp.uint32(19))
    x1 = x1 ^ x0
    for i in range(5):
        for r in rots[i % 2][(1 if i == 0 else 0):]:
            x0 = x0 + x1
            x1 = (x1 << jnp.uint32(r)) | (x1 >> jnp.uint32(32 - r))
            x1 = x1 ^ x0
        if c0[i]:
            x0 = x0 + jnp.uint32(c0[i])
        x1 = x1 + jnp.uint32(c1[i])
    return x0 ^ x1


def _sc_bits_kernel(out_hbm, rowbuf, sem):
    nc = 2
    rpw = _ROWS // (nc * 16)
    wid = lax.axis_index("s") * nc + lax.axis_index("c")
    iota16 = lax.broadcasted_iota(jnp.uint32, (16,), 0)

    copies = []
    for k in range(rpw):
        r = wid * rpw + k
        base = (r * _COLS + _C0 + _K1).astype(jnp.uint32)
        b = k % 2
        if k >= 2:
            copies[k - 2].wait()

        def body(i, c):
            x1 = iota16 + (base + (i * 16).astype(jnp.uint32))
            rowbuf[b, pl.ds(i * 16, 16)] = _threefry_from_x1(x1)
            return c

        lax.fori_loop(0, (_COLS - _C0) // 16, body, 0, unroll=8)
        copies.append(pltpu.async_copy(rowbuf.at[b], out_hbm.at[r], sem))
    copies[-2].wait()
    copies[-1].wait()


def _sc_stripe_bits():
    mesh = plsc.VectorSubcoreMesh(core_axis_name="c", subcore_axis_name="s")
    return pl.kernel(
        _sc_bits_kernel,
        mesh=mesh,
        out_type=jax.ShapeDtypeStruct((_ROWS, _SW), jnp.uint32),
        scratch_types=[
            pltpu.VMEM((2, _SW), jnp.uint32),
            pltpu.SemaphoreType.DMA,
        ],
    )()


def _main_body(q_ref, base_ref, s_ref, best_ref, bidx_ref):
    j = pl.program_id(0)

    @pl.when(j == 0)
    def _init():
        s_ref[...] = jnp.zeros_like(s_ref)
        best_ref[...] = jnp.full_like(best_ref, 1.0)
        bidx_ref[...] = jnp.zeros_like(bidx_ref)

    q = q_ref[...]
    x1 = base_ref[...] + (j * _W).astype(jnp.uint32)

    bits = _threefry_from_x1(x1)
    fb = (bits >> jnp.uint32(9)) | jnp.uint32(0x3F800000)
    lu = jnp.log(jax.lax.bitcast_convert_type(fb, jnp.float32) - 1.0)
    e = jnp.exp(q)
    t = e / lu
    xi = jax.lax.bitcast_convert_type(x1, jnp.int32)

    ls = s_ref[...]
    lb = best_ref[...]
    li = bidx_ref[...]
    for k in range(_W // 128):
        sl = slice(k * 128, (k + 1) * 128)
        tc = t[:, sl]
        ls = ls + e[:, sl]
        upd = tc < lb
        lb = jnp.where(upd, tc, lb)
        li = jnp.where(upd, xi[:, sl], li)
    s_ref[...] = ls
    best_ref[...] = lb
    bidx_ref[...] = li


def _stripe_body(q_ref, bits_ref, base_ref, end_ref, s_in, b_in, i_in,
                 s_ref, best_ref, bidx_ref):
    j = pl.program_id(0)

    @pl.when(j == 0)
    def _init():
        s_ref[...] = s_in[...]
        best_ref[...] = b_in[...]
        bidx_ref[...] = i_in[...]

    q = q_ref[...]
    x1 = base_ref[...] + ((_NCM + j) * _W).astype(jnp.uint32)
    bits = bits_ref[...]
    fb = (bits >> jnp.uint32(9)) | jnp.uint32(0x3F800000)
    lu = jnp.log(jax.lax.bitcast_convert_type(fb, jnp.float32) - 1.0)
    e = jnp.exp(q)
    t = e / lu

    valid = x1 < end_ref[...]
    e = jnp.where(valid, e, 0.0)
    t = jnp.where(valid, t, 1.0)
    xi = jax.lax.bitcast_convert_type(x1, jnp.int32)

    ls = s_ref[...]
    lb = best_ref[...]
    li = bidx_ref[...]
    for k in range(_W // 128):
        sl = slice(k * 128, (k + 1) * 128)
        tc = t[:, sl]
        ls = ls + e[:, sl]
        upd = tc < lb
        lb = jnp.where(upd, tc, lb)
        li = jnp.where(upd, xi[:, sl], li)
    s_ref[...] = ls
    best_ref[...] = lb
    bidx_ref[...] = li


def _final_body(s_ref, best_ref, bidx_ref, act_ref, logp_ref):
    ls = s_ref[...]
    lb = best_ref[...]
    li = bidx_ref[...]
    s_tot = jnp.sum(ls, axis=1, keepdims=True)
    m = jnp.min(lb, axis=1, keepdims=True)
    sel = lb == m
    big = jnp.int32(2**31 - 1)
    fli = jnp.min(jnp.where(sel, li, big), axis=1, keepdims=True)

    x1w = jnp.broadcast_to(
        jax.lax.bitcast_convert_type(fli, jnp.uint32), best_ref.shape
    )
    bits = _threefry_from_x1(x1w)
    fb = (bits >> jnp.uint32(9)) | jnp.uint32(0x3F800000)
    luw = jnp.log(jax.lax.bitcast_convert_type(fb, jnp.float32) - 1.0)
    qw = jnp.log(m * luw[:, :1])

    rowoff = jax.lax.broadcasted_iota(jnp.int32, (_ROWS, 1), 0) * _COLS
    idx = fli - rowoff - _K1
    act_ref[...] = jnp.broadcast_to(idx, act_ref.shape)
    logp_ref[...] = jnp.broadcast_to(qw - jnp.log(s_tot), logp_ref.shape)


def kernel(q):
    part = jax.ShapeDtypeStruct((_ROWS, 128), jnp.float32)
    parti = jax.ShapeDtypeStruct((_ROWS, 128), jnp.int32)
    base_c = jnp.asarray(_X1_BASE)
    end_c = jnp.asarray(_X1_END)

    sc_bits = _sc_stripe_bits()

    acc_spec = pl.BlockSpec((_ROWS, 128), lambda c: (0, 0))
    const_spec = pl.BlockSpec((_ROWS, _W), lambda c: (0, 0))
    s_p, best_p, bidx_p = pl.pallas_call(
        _main_body,
        grid=(_NCM,),
        in_specs=[
            pl.BlockSpec((_ROWS, _W), lambda c: (0, c)),
            const_spec,
        ],
        out_specs=[acc_spec, acc_spec, acc_spec],
        out_shape=[part, part, parti],
        compiler_params=pltpu.CompilerParams(
            dimension_semantics=("arbitrary",),
        ),
    )(q, base_c)

    s_m, best_m, bidx_m = pl.pallas_call(
        _stripe_body,
        grid=(_NCS,),
        in_specs=[
            pl.BlockSpec((_ROWS, _W), lambda c: (0, _NCM + c)),
            pl.BlockSpec((_ROWS, _W), lambda c: (0, c)),
            const_spec,
            const_spec,
            acc_spec,
            acc_spec,
            acc_spec,
        ],
        out_specs=[acc_spec, acc_spec, acc_spec],
        out_shape=[part, part, parti],
        compiler_params=pltpu.CompilerParams(
            dimension_semantics=("arbitrary",),
        ),
    )(q, sc_bits, base_c, end_c, s_p, best_p, bidx_p)

    full = pl.BlockSpec((_ROWS, 128), lambda: (0, 0))
    act, logp = pl.pallas_call(
        _final_body,
        in_specs=[full, full, full],
        out_specs=[full, full],
        out_shape=[parti, part],
    )(s_m, best_m, bidx_m)

    pi_action = act[:, :1].astype(jnp.int64)
    logp_pi = logp[:, :1]
    return (pi_action, logp_pi)


# speedup vs baseline: 1.0008x; 1.0008x over previous
import numpy as np

import jax
import jax.numpy as jnp
from jax import lax
from jax.experimental import pallas as pl
from jax.experimental.pallas import tpu as pltpu
from jax.experimental.pallas import tpu_sc as plsc

_ROWS = 128
_COLS = 100000
_W = 2048
_NCM = 33
_C0 = _NCM * _W
_SW = 32768
_NCS = _SW // _W
_K1 = 42

_X1_BASE = (
    np.arange(_ROWS, dtype=np.uint32)[:, None] * np.uint32(_COLS)
    + np.arange(_W, dtype=np.uint32)[None, :]
    + np.uint32(_K1)
)
_X1_END = np.broadcast_to(
    (np.arange(_ROWS, dtype=np.uint32)[:, None] + 1) * np.uint32(_COLS)
    + np.uint32(_K1),
    (_ROWS, _W),
).copy()


def _threefry_from_x1(x1):
    ks2 = 0x1BD11BDA ^ _K1
    rots = ((13, 15, 26, 6), (17, 29, 16, 24))
    c0 = (_K1, ks2, 0, _K1, ks2)
    c1 = (ks2 + 1, 2, _K1 + 3, ks2 + 4, 5)

    x0 = x1
    x1 = (x1 << jnp.uint32(13)) | (x1 >> jnp.uint32(19))
    x1 = x1 ^ x0
    for i in range(5):
        for r in rots[i % 2][(1 if i == 0 else 0):]:
            x0 = x0 + x1
            x1 = (x1 << jnp.uint32(r)) | (x1 >> jnp.uint32(32 - r))
            x1 = x1 ^ x0
        if c0[i]:
            x0 = x0 + jnp.uint32(c0[i])
        x1 = x1 + jnp.uint32(c1[i])
    return x0 ^ x1


def _sc_bits_kernel(out_hbm, rowbuf, sem):
    nc = 2
    rpw = _ROWS // (nc * 16)
    wid = lax.axis_index("s") * nc + lax.axis_index("c")
    iota16 = lax.broadcasted_iota(jnp.uint32, (16,), 0)

    copies = []
    for k in range(rpw):
        r = wid * rpw + k
        base = (r * _COLS + _C0 + _K1).astype(jnp.uint32)
        b = k % 2
        if k >= 2:
            copies[k - 2].wait()

        def body(i, c):
            x1 = iota16 + (base + (i * 16).astype(jnp.uint32))
            rowbuf[b, pl.ds(i * 16, 16)] = _threefry_from_x1(x1)
            return c

        lax.fori_loop(0, (_COLS - _C0) // 16, body, 0, unroll=16)
        copies.append(pltpu.async_copy(rowbuf.at[b], out_hbm.at[r], sem))
    copies[-2].wait()
    copies[-1].wait()


def _sc_stripe_bits():
    mesh = plsc.VectorSubcoreMesh(core_axis_name="c", subcore_axis_name="s")
    return pl.kernel(
        _sc_bits_kernel,
        mesh=mesh,
        out_type=jax.ShapeDtypeStruct((_ROWS, _SW), jnp.uint32),
        scratch_types=[
            pltpu.VMEM((2, _SW), jnp.uint32),
            pltpu.SemaphoreType.DMA,
        ],
    )()


def _main_body(q_ref, base_ref, s_ref, best_ref, bidx_ref):
    j = pl.program_id(0)

    @pl.when(j == 0)
    def _init():
        s_ref[...] = jnp.zeros_like(s_ref)
        best_ref[...] = jnp.full_like(best_ref, 1.0)
        bidx_ref[...] = jnp.zeros_like(bidx_ref)

    q = q_ref[...]
    x1 = base_ref[...] + (j * _W).astype(jnp.uint32)

    bits = _threefry_from_x1(x1)
    fb = (bits >> jnp.uint32(9)) | jnp.uint32(0x3F800000)
    lu = jnp.log(jax.lax.bitcast_convert_type(fb, jnp.float32) - 1.0)
    e = jnp.exp(q)
    t = e / lu
    xi = jax.lax.bitcast_convert_type(x1, jnp.int32)

    ls = s_ref[...]
    lb = best_ref[...]
    li = bidx_ref[...]
    for k in range(_W // 128):
        sl = slice(k * 128, (k + 1) * 128)
        tc = t[:, sl]
        ls = ls + e[:, sl]
        upd = tc < lb
        lb = jnp.where(upd, tc, lb)
        li = jnp.where(upd, xi[:, sl], li)
    s_ref[...] = ls
    best_ref[...] = lb
    bidx_ref[...] = li


def _stripe_body(q_ref, bits_ref, base_ref, end_ref, s_in, b_in, i_in,
                 s_ref, best_ref, bidx_ref):
    j = pl.program_id(0)

    @pl.when(j == 0)
    def _init():
        s_ref[...] = s_in[...]
        best_ref[...] = b_in[...]
        bidx_ref[...] = i_in[...]

    q = q_ref[...]
    x1 = base_ref[...] + ((_NCM + j) * _W).astype(jnp.uint32)
    bits = bits_ref[...]
    fb = (bits >> jnp.uint32(9)) | jnp.uint32(0x3F800000)
    lu = jnp.log(jax.lax.bitcast_convert_type(fb, jnp.float32) - 1.0)
    e = jnp.exp(q)
    t = e / lu

    valid = x1 < end_ref[...]
    e = jnp.where(valid, e, 0.0)
    t = jnp.where(valid, t, 1.0)
    xi = jax.lax.bitcast_convert_type(x1, jnp.int32)

    ls = s_ref[...]
    lb = best_ref[...]
    li = bidx_ref[...]
    for k in range(_W // 128):
        sl = slice(k * 128, (k + 1) * 128)
        tc = t[:, sl]
        ls = ls + e[:, sl]
        upd = tc < lb
        lb = jnp.where(upd, tc, lb)
        li = jnp.where(upd, xi[:, sl], li)
    s_ref[...] = ls
    best_ref[...] = lb
    bidx_ref[...] = li


def _final_body(s_ref, best_ref, bidx_ref, act_ref, logp_ref):
    ls = s_ref[...]
    lb = best_ref[...]
    li = bidx_ref[...]
    s_tot = jnp.sum(ls, axis=1, keepdims=True)
    m = jnp.min(lb, axis=1, keepdims=True)
    sel = lb == m
    big = jnp.int32(2**31 - 1)
    fli = jnp.min(jnp.where(sel, li, big), axis=1, keepdims=True)

    x1w = jnp.broadcast_to(
        jax.lax.bitcast_convert_type(fli, jnp.uint32), best_ref.shape
    )
    bits = _threefry_from_x1(x1w)
    fb = (bits >> jnp.uint32(9)) | jnp.uint32(0x3F800000)
    luw = jnp.log(jax.lax.bitcast_convert_type(fb, jnp.float32) - 1.0)
    qw = jnp.log(m * luw[:, :1])

    rowoff = jax.lax.broadcasted_iota(jnp.int32, (_ROWS, 1), 0) * _COLS
    idx = fli - rowoff - _K1
    act_ref[...] = jnp.broadcast_to(idx, act_ref.shape)
    logp_ref[...] = jnp.broadcast_to(qw - jnp.log(s_tot), logp_ref.shape)


def kernel(q):
    part = jax.ShapeDtypeStruct((_ROWS, 128), jnp.float32)
    parti = jax.ShapeDtypeStruct((_ROWS, 128), jnp.int32)
    base_c = jnp.asarray(_X1_BASE)
    end_c = jnp.asarray(_X1_END)

    sc_bits = _sc_stripe_bits()

    acc_spec = pl.BlockSpec((_ROWS, 128), lambda c: (0, 0))
    const_spec = pl.BlockSpec((_ROWS, _W), lambda c: (0, 0))
    s_p, best_p, bidx_p = pl.pallas_call(
        _main_body,
        grid=(_NCM,),
        in_specs=[
            pl.BlockSpec((_ROWS, _W), lambda c: (0, c)),
            const_spec,
        ],
        out_specs=[acc_spec, acc_spec, acc_spec],
        out_shape=[part, part, parti],
        compiler_params=pltpu.CompilerParams(
            dimension_semantics=("arbitrary",),
        ),
    )(q, base_c)

    s_m, best_m, bidx_m = pl.pallas_call(
        _stripe_body,
        grid=(_NCS,),
        in_specs=[
            pl.BlockSpec((_ROWS, _W), lambda c: (0, _NCM + c)),
            pl.BlockSpec((_ROWS, _W), lambda c: (0, c)),
            const_spec,
            const_spec,
            acc_spec,
            acc_spec,
            acc_spec,
        ],
        out_specs=[acc_spec, acc_spec, acc_spec],
        out_shape=[part, part, parti],
        compiler_params=pltpu.CompilerParams(
            dimension_semantics=("arbitrary",),
        ),
    )(q, sc_bits, base_c, end_c, s_p, best_p, bidx_p)

    full = pl.BlockSpec((_ROWS, 128), lambda: (0, 0))
    act, logp = pl.pallas_call(
        _final_body,
        in_specs=[full, full, full],
        out_specs=[full, full],
        out_shape=[parti, part],
    )(s_m, best_m, bidx_m)

    pi_action = act[:, :1].astype(jnp.int64)
    logp_pi = logp[:, :1]
    return (pi_action, logp_pi)
